# baseline (speedup 1.0000x reference)
import functools

import numpy as np
import jax
import jax.numpy as jnp
from jax.experimental import pallas as pl
from jax.experimental.pallas import tpu as pltpu

_RATIO = 0.6
_RATE = 0.5
_FILL_VALUE = 1


def _make_grid_mask_np(H, W, ratio, rng):
    mask_size = int(max(H, W) * 2)
    lo = int(min(H * 0.5, W * 0.3))
    hi = int(max(H * 0.5, W * 0.3)) + 1
    gridblock = int(rng.integers(lo, hi))
    if ratio == 1:
        length = int(rng.integers(1, gridblock + 1))
    else:
        length = int(min(max(int(gridblock * ratio + 0.5), 1), gridblock - 1))
    mask = np.zeros((mask_size, mask_size), dtype=np.int32)
    for _ in range(2):
        start_w = int(rng.integers(0, gridblock + 1))
        for i in range(mask_size // gridblock):
            start = gridblock * i + start_w
            end = min(start + length, mask_size)
            if end > start:
                mask[start:end, :] = _FILL_VALUE
        mask = mask.T.copy()
    top = (mask_size - H) // 2
    left = (mask_size - W) // 2
    return mask[top:top + H, left:left + W]


@functools.lru_cache(maxsize=None)
def _mask_factors(B, H, W, C):
    rng = np.random.default_rng(0)
    masks = []
    for _ in range(B):
        m = _make_grid_mask_np(H, W, _RATIO, rng)
        rate_cond = rng.random() < _RATE
        if not rate_cond:
            m = np.ones((H, W), dtype=np.int32)
        masks.append(m)
    masks = np.stack(masks).astype(np.float32)
    row = masks.min(axis=2)
    col = masks.min(axis=1)
    rec = np.maximum(row[:, :, None], col[:, None, :])
    assert np.array_equal(rec, masks), "mask not row/col separable"
    col3 = np.repeat(col, C, axis=1)
    return row[:, :, None], col3[:, None, :]


def _body(row_ref, col_ref, img_ref, out_ref):
    m = jnp.maximum(row_ref[...], col_ref[...])
    out_ref[...] = img_ref[...] * m


def kernel(images):
    B, H, W, C = images.shape
    row, col3 = _mask_factors(B, H, W, C)
    row = jnp.asarray(row)
    col3 = jnp.asarray(col3)
    WC = W * C
    img2 = images.reshape(B, H, WC)

    HBLK = 256
    grid = (B, H // HBLK)
    out = pl.pallas_call(
        _body,
        grid=grid,
        in_specs=[
            pl.BlockSpec((1, HBLK, 1), lambda b, h: (b, h, 0)),
            pl.BlockSpec((1, 1, WC), lambda b, h: (b, 0, 0)),
            pl.BlockSpec((1, HBLK, WC), lambda b, h: (b, h, 0)),
        ],
        out_specs=pl.BlockSpec((1, HBLK, WC), lambda b, h: (b, h, 0)),
        out_shape=jax.ShapeDtypeStruct((B, H, WC), jnp.float32),
        compiler_params=pltpu.CompilerParams(
            dimension_semantics=("parallel", "parallel"),
        ),
    )(row, col3, img2)
    return out.reshape(B, H, W, C)


# speedup vs baseline: 6.2051x; 6.2051x over previous
import functools

import numpy as np
import jax
import jax.numpy as jnp
from jax.experimental import pallas as pl
from jax.experimental.pallas import tpu as pltpu

_RATIO = 0.6
_RATE = 0.5
_FILL_VALUE = 1
_LANES = 128
_SUBLANES = 8


def _make_grid_mask_np(H, W, ratio, rng):
    mask_size = int(max(H, W) * 2)
    lo = int(min(H * 0.5, W * 0.3))
    hi = int(max(H * 0.5, W * 0.3)) + 1
    gridblock = int(rng.integers(lo, hi))
    if ratio == 1:
        length = int(rng.integers(1, gridblock + 1))
    else:
        length = int(min(max(int(gridblock * ratio + 0.5), 1), gridblock - 1))
    mask = np.zeros((mask_size, mask_size), dtype=np.int32)
    for _ in range(2):
        start_w = int(rng.integers(0, gridblock + 1))
        for i in range(mask_size // gridblock):
            start = gridblock * i + start_w
            end = min(start + length, mask_size)
            if end > start:
                mask[start:end, :] = _FILL_VALUE
        mask = mask.T.copy()
    top = (mask_size - H) // 2
    left = (mask_size - W) // 2
    return mask[top:top + H, left:left + W]


@functools.lru_cache(maxsize=None)
def _mask_factors(B, H, W):
    rng = np.random.default_rng(0)
    masks = []
    for _ in range(B):
        m = _make_grid_mask_np(H, W, _RATIO, rng)
        rate_cond = rng.random() < _RATE
        if not rate_cond:
            m = np.ones((H, W), dtype=np.int32)
        masks.append(m)
    masks = np.stack(masks).astype(np.float32)
    row = masks.min(axis=2)
    col = masks.min(axis=1)
    rec = np.maximum(row[:, :, None], col[:, None, :])
    assert np.array_equal(rec, masks), "mask not row/col separable"
    rowb = np.repeat(row[:, :, None], _LANES, axis=2)
    colb = np.repeat(col[:, None, :], _SUBLANES, axis=1)
    return rowb, colb


def _body(row_ref, col_ref, img_ref, out_ref):
    H = row_ref.shape[1]
    W = col_ref.shape[2]
    C = img_ref.shape[1]
    rowb = row_ref[0]
    colb = col_ref[0]
    row_full = jnp.concatenate([rowb] * (W // _LANES), axis=1)
    col_full = jnp.concatenate([colb] * (H // _SUBLANES), axis=0)
    m = jnp.maximum(row_full, col_full)
    for c in range(C):
        out_ref[0, c] = img_ref[0, c] * m


def kernel(images):
    B, H, W, C = images.shape
    rowb, colb = _mask_factors(B, H, W)
    rowb = jnp.asarray(rowb)
    colb = jnp.asarray(colb)
    img_t = jnp.transpose(images, (0, 3, 1, 2))

    grid = (B,)
    out = pl.pallas_call(
        _body,
        grid=grid,
        in_specs=[
            pl.BlockSpec((1, H, _LANES), lambda b: (b, 0, 0)),
            pl.BlockSpec((1, _SUBLANES, W), lambda b: (b, 0, 0)),
            pl.BlockSpec((1, C, H, W), lambda b: (b, 0, 0, 0)),
        ],
        out_specs=pl.BlockSpec((1, C, H, W), lambda b: (b, 0, 0, 0)),
        out_shape=jax.ShapeDtypeStruct((B, C, H, W), jnp.float32),
        compiler_params=pltpu.CompilerParams(
            dimension_semantics=("parallel",),
        ),
    )(rowb, colb, img_t)
    return jnp.transpose(out, (0, 2, 3, 1))


# speedup vs baseline: 6.3477x; 1.0230x over previous
import functools

import numpy as np
import jax
import jax.numpy as jnp
from jax.experimental import pallas as pl
from jax.experimental.pallas import tpu as pltpu

_RATIO = 0.6
_RATE = 0.5
_FILL_VALUE = 1
_LANES = 128
_SUBLANES = 8


def _make_grid_mask_np(H, W, ratio, rng):
    mask_size = int(max(H, W) * 2)
    lo = int(min(H * 0.5, W * 0.3))
    hi = int(max(H * 0.5, W * 0.3)) + 1
    gridblock = int(rng.integers(lo, hi))
    if ratio == 1:
        length = int(rng.integers(1, gridblock + 1))
    else:
        length = int(min(max(int(gridblock * ratio + 0.5), 1), gridblock - 1))
    mask = np.zeros((mask_size, mask_size), dtype=np.int32)
    for _ in range(2):
        start_w = int(rng.integers(0, gridblock + 1))
        for i in range(mask_size // gridblock):
            start = gridblock * i + start_w
            end = min(start + length, mask_size)
            if end > start:
                mask[start:end, :] = _FILL_VALUE
        mask = mask.T.copy()
    top = (mask_size - H) // 2
    left = (mask_size - W) // 2
    return mask[top:top + H, left:left + W]


@functools.lru_cache(maxsize=None)
def _mask_factors(B, H, W):
    rng = np.random.default_rng(0)
    masks = []
    for _ in range(B):
        m = _make_grid_mask_np(H, W, _RATIO, rng)
        rate_cond = rng.random() < _RATE
        if not rate_cond:
            m = np.ones((H, W), dtype=np.int32)
        masks.append(m)
    masks = np.stack(masks).astype(np.float32)
    row = masks.min(axis=2)
    col = masks.min(axis=1)
    rec = np.maximum(row[:, :, None], col[:, None, :])
    assert np.array_equal(rec, masks), "mask not row/col separable"
    rowb = np.repeat(row[:, :, None], _LANES, axis=2).astype(np.int8)
    colb = np.repeat(col[:, None, :], _SUBLANES, axis=1).astype(np.int8)
    return rowb, colb


def _body(row_ref, col_ref, img_ref, out_ref):
    H = row_ref.shape[1]
    W = col_ref.shape[2]
    C = img_ref.shape[1]
    rowb = row_ref[0]
    colb = col_ref[0]
    row_full = jnp.concatenate([rowb] * (W // _LANES), axis=1)
    col_full = jnp.concatenate([colb] * (H // _SUBLANES), axis=0)
    m = (row_full | col_full).astype(jnp.float32)
    for c in range(C):
        out_ref[0, c] = img_ref[0, c] * m


def kernel(images):
    B, H, W, C = images.shape
    rowb, colb = _mask_factors(B, H, W)
    rowb = jnp.asarray(rowb)
    colb = jnp.asarray(colb)
    img_t = jnp.transpose(images, (0, 3, 1, 2))

    grid = (B,)
    out = pl.pallas_call(
        _body,
        grid=grid,
        in_specs=[
            pl.BlockSpec((1, H, _LANES), lambda b: (b, 0, 0)),
            pl.BlockSpec((1, _SUBLANES, W), lambda b: (b, 0, 0)),
            pl.BlockSpec((1, C, H, W), lambda b: (b, 0, 0, 0)),
        ],
        out_specs=pl.BlockSpec((1, C, H, W), lambda b: (b, 0, 0, 0)),
        out_shape=jax.ShapeDtypeStruct((B, C, H, W), jnp.float32),
        compiler_params=pltpu.CompilerParams(
            dimension_semantics=("parallel",),
        ),
    )(rowb, colb, img_t)
    return jnp.transpose(out, (0, 2, 3, 1))
